# Initial kernel scaffold; baseline (speedup 1.0000x reference)
import jax
import jax.numpy as jnp
from jax.experimental import pallas as pl


def kernel(keys, points, feats, w1, b1, w2, b2, w3, b3, f1, fb1, f2, fb2):
    raise NotImplementedError("write your pallas kernel here")


# speedup vs baseline: 5.7459x; 5.7459x over previous
import functools

import jax
import jax.numpy as jnp
from jax import lax
from jax.experimental import pallas as pl
from jax.experimental.pallas import tpu as pltpu
from jax.experimental.pallas import tpu_sc as plsc

B, K, N = 4, 1024, 4096
DIM, CIN, NB, CMID, COUT = 3, 64, 32, 16, 128

RA = 512
RC = 256
CHUNK = 128


def _topk_body(pts_ref, keys_ref, idx_ref, dist_ref):
    g = pl.program_id(0)
    b = g // (K // RA)
    px = pts_ref[0, :, 0:1]
    py = pts_ref[0, :, 1:2]
    pz = pts_ref[0, :, 2:3]
    kx = keys_ref[0, 0:1, :]
    ky = keys_ref[0, 1:2, :]
    kz = keys_ref[0, 2:3, :]
    dx = px - kx
    dy = py - ky
    dz = pz - kz
    dist_ref[...] = dx * dx + dy * dy + dz * dz
    iota_n = lax.broadcasted_iota(jnp.int32, (N, RA), 0)
    base = b * N
    for t in range(NB):
        d = dist_ref[...]
        m = jnp.min(d, axis=0, keepdims=True)
        idx = jnp.min(jnp.where(d == m, iota_n, N), axis=0, keepdims=True)
        idx_ref[0, t : t + 1, :] = idx + base
        dist_ref[...] = jnp.where(iota_n == idx, jnp.inf, d)


def _topk_indices(points, keys_t):
    grid = (B * K) // RA
    return pl.pallas_call(
        _topk_body,
        grid=(grid,),
        in_specs=[
            pl.BlockSpec((1, N, DIM), lambda g: (g // (K // RA), 0, 0)),
            pl.BlockSpec((1, DIM, RA), lambda g: (g // (K // RA), 0, g % (K // RA))),
        ],
        out_specs=pl.BlockSpec((1, NB, RA), lambda g: (g // (K // RA), 0, g % (K // RA))),
        out_shape=jax.ShapeDtypeStruct((B, NB, K), jnp.int32),
        scratch_shapes=[pltpu.VMEM((N, RA), jnp.float32)],
    )(points, keys_t)


def _make_gather():
    info = plsc.get_sparse_core_info()
    NC, NS = info.num_cores, info.num_subcores
    NW = NC * NS
    total = B * NB * K
    per_w = total // NW
    n_chunks = per_w // CHUNK
    mesh = plsc.VectorSubcoreMesh(core_axis_name="c", subcore_axis_name="s")

    @functools.partial(
        pl.kernel,
        mesh=mesh,
        out_type=jax.ShapeDtypeStruct((total, 128), jnp.float32),
        scratch_types=[
            pltpu.VMEM((per_w,), jnp.int32),
            pltpu.VMEM((CHUNK, 128), jnp.float32),
            pltpu.SemaphoreType.DMA,
        ],
    )
    def gather(tbl_hbm, idx_hbm, rows_hbm, idx_v, buf, sem1):
        wid = lax.axis_index("s") * NC + lax.axis_index("c")
        base = wid * per_w
        pltpu.sync_copy(idx_hbm.at[pl.ds(base, per_w)], idx_v)

        def body(ci, _):
            off = ci * CHUNK
            isl = idx_v.at[pl.ds(off, CHUNK)]
            pltpu.async_copy(tbl_hbm.at[isl], buf, sem1).wait()
            pltpu.sync_copy(buf, rows_hbm.at[pl.ds(base + off, CHUNK)])
            return ()

        lax.fori_loop(0, n_chunks, body, ())

    return gather


_gather_rows = None


def _gather(tbl, idx_flat):
    global _gather_rows
    if _gather_rows is None:
        _gather_rows = _make_gather()
    return _gather_rows(tbl, idx_flat)


def _dense_body(rows_ref, keys_ref, w1_ref, b1_ref, w2_ref, b2_ref,
                w3_ref, b3_ref, f1_ref, fb1_ref, f2_ref, fb2_ref, out_ref,
                e_ref):
    f = rows_ref[0, :, :, 0:CIN]
    p = rows_ref[0, :, :, CIN : CIN + DIM]
    k3 = keys_ref[0]
    rel = p - k3[None, :, :]
    x = rel.reshape(NB * RC, DIM)
    h1 = jax.nn.relu(
        jnp.dot(x, w1_ref[...], preferred_element_type=jnp.float32)
        + b1_ref[...])
    h2 = jax.nn.relu(
        jnp.dot(h1, w2_ref[...], preferred_element_type=jnp.float32)
        + b2_ref[...])
    m = (jnp.dot(h2, w3_ref[...], preferred_element_type=jnp.float32)
         + b3_ref[...])
    m3 = m.reshape(NB, RC, CMID)
    e_ref[...] = jnp.zeros((RC, CMID, CIN), jnp.float32)
    for t in range(NB):
        e_ref[...] += m3[t][:, :, None] * f[t][:, None, :]
    e_flat = e_ref[...].reshape(RC, CMID * CIN)
    h = jax.nn.relu(
        jnp.dot(e_flat, f1_ref[...], preferred_element_type=jnp.float32)
        + fb1_ref[...])
    out_ref[0] = (jnp.dot(h, f2_ref[...], preferred_element_type=jnp.float32)
                  + fb2_ref[...])


def _dense(rows, keys, w1, b1, w2, b2, w3, b3, f1, fb1, f2, fb2):
    grid = (B * K) // RC
    kb = K // RC
    full = lambda g: (0, 0)
    return pl.pallas_call(
        _dense_body,
        grid=(grid,),
        in_specs=[
            pl.BlockSpec((1, NB, RC, 128), lambda g: (g // kb, 0, g % kb, 0)),
            pl.BlockSpec((1, RC, DIM), lambda g: (g // kb, g % kb, 0)),
            pl.BlockSpec((DIM, 32), full),
            pl.BlockSpec((1, 32), full),
            pl.BlockSpec((32, 32), full),
            pl.BlockSpec((1, 32), full),
            pl.BlockSpec((32, CMID), full),
            pl.BlockSpec((1, CMID), full),
            pl.BlockSpec((CIN * CMID, 256), full),
            pl.BlockSpec((1, 256), full),
            pl.BlockSpec((256, COUT), full),
            pl.BlockSpec((1, COUT), full),
        ],
        out_specs=pl.BlockSpec((1, RC, COUT), lambda g: (g // kb, g % kb, 0)),
        out_shape=jax.ShapeDtypeStruct((B, K, COUT), jnp.float32),
        scratch_shapes=[pltpu.VMEM((RC, CMID, CIN), jnp.float32)],
    )(rows, keys, w1, b1, w2, b2, w3, b3, f1, fb1, f2, fb2)


def kernel(keys, points, feats, w1, b1, w2, b2, w3, b3, f1, fb1, f2, fb2):
    keys_t = jnp.swapaxes(keys, 1, 2)
    idx = _topk_indices(points, keys_t)
    idx_flat = idx.reshape(-1)

    tbl = jnp.concatenate(
        [feats.reshape(B * N, CIN),
         points.reshape(B * N, DIM),
         jnp.zeros((B * N, 128 - CIN - DIM), jnp.float32)], axis=1)
    rows_flat = _gather(tbl, idx_flat)
    rows = rows_flat.reshape(B, NB, K, 128)

    out = _dense(
        rows, keys,
        w1, b1.reshape(1, -1), w2, b2.reshape(1, -1), w3, b3.reshape(1, -1),
        f1, fb1.reshape(1, -1), f2, fb2.reshape(1, -1))
    return out
